# speedup vs baseline: 1.0435x; 1.0435x over previous
import jax
import jax.numpy as jnp
from jax.experimental import pallas as pl
from jax.experimental.pallas import tpu as pltpu

_HIDDEN = 2048
_INTER = 1536
_E = 16
_JITTER = 0.01
_FCHUNK = 512
_NC = _INTER // _FCHUNK


def _moe_body(x_ref, gw_ref, w1_ref, w3_ref, w2_ref, out_ref, logits_ref,
              comb_ref):
    e = pl.program_id(0)
    c = pl.program_id(1)

    @pl.when((e == 0) & (c == 0))
    def _routing():
        x = x_ref[...]
        scores = jax.lax.dot_general(
            x, gw_ref[...], (((1,), (1,)), ((), ())),
            preferred_element_type=jnp.float32)
        logits_ref[...] = scores
        tdim, edim = scores.shape
        eio = jax.lax.broadcasted_iota(jnp.int32, (tdim, edim), 1)
        neg = jnp.float32(-jnp.inf)
        m1 = jnp.max(scores, axis=1, keepdims=True)
        sel0 = jnp.min(jnp.where(scores == m1, eio, edim), axis=1,
                       keepdims=True)
        scores2 = jnp.where(eio == sel0, neg, scores)
        m2 = jnp.max(scores2, axis=1, keepdims=True)
        sel1 = jnp.min(jnp.where(scores2 == m2, eio, edim), axis=1,
                       keepdims=True)
        f1 = jnp.maximum(jnp.abs(scores), m1)
        mask1 = ((m1 - scores) / f1) > (2.0 * _JITTER)
        d1 = jnp.sum(jnp.exp(jnp.where(mask1, neg, scores) - m1), axis=1,
                     keepdims=True)
        f2 = jnp.maximum(jnp.abs(scores), m2)
        mask2 = (((m2 - scores) / f2) > (2.0 * _JITTER)) | (eio == sel0)
        d2 = jnp.sum(jnp.exp(jnp.where(mask2, neg, scores) - m2), axis=1,
                     keepdims=True)
        comb_ref[...] = (jnp.where(eio == sel0, 1.0 / d1, 0.0) +
                         jnp.where(eio == sel1, 1.0 / d2, 0.0))
        out_ref[...] = jnp.zeros_like(out_ref)

    x = x_ref[...]
    w1 = w1_ref[0]
    w3 = w3_ref[0]
    w2 = w2_ref[0]
    h1 = jax.lax.dot_general(x, w1, (((1,), (1,)), ((), ())),
                             preferred_element_type=jnp.float32)
    h3 = jax.lax.dot_general(x, w3, (((1,), (1,)), ((), ())),
                             preferred_element_type=jnp.float32)
    act = (h1 * jax.lax.logistic(h1)) * h3
    outp = jax.lax.dot_general(act, w2, (((1,), (1,)), ((), ())),
                               preferred_element_type=jnp.float32)
    eio = jax.lax.broadcasted_iota(jnp.int32, comb_ref.shape, 1)
    col = jnp.sum(jnp.where(eio == e, comb_ref[...], 0.0), axis=1,
                  keepdims=True)
    out_ref[...] += outp * col


def kernel(hidden_states, gate_w, w1, w2, w3):
    bsz, seq, hdim = hidden_states.shape
    tdim = bsz * seq
    x = hidden_states.reshape(tdim, hdim)

    out, logits = pl.pallas_call(
        _moe_body,
        grid=(_E, _NC),
        in_specs=[
            pl.BlockSpec((tdim, _HIDDEN), lambda e, c: (0, 0)),
            pl.BlockSpec((_E, _HIDDEN), lambda e, c: (0, 0)),
            pl.BlockSpec((1, _FCHUNK, _HIDDEN), lambda e, c: (e, c, 0)),
            pl.BlockSpec((1, _FCHUNK, _HIDDEN), lambda e, c: (e, c, 0)),
            pl.BlockSpec((1, _HIDDEN, _FCHUNK), lambda e, c: (e, 0, c)),
        ],
        out_specs=[
            pl.BlockSpec((tdim, _HIDDEN), lambda e, c: (0, 0)),
            pl.BlockSpec((tdim, _E), lambda e, c: (0, 0)),
        ],
        out_shape=[
            jax.ShapeDtypeStruct((tdim, _HIDDEN), jnp.float32),
            jax.ShapeDtypeStruct((tdim, _E), jnp.float32),
        ],
        scratch_shapes=[pltpu.VMEM((tdim, _E), jnp.float32)],
    )(x, gate_w, w1, w3, w2)

    return out.reshape(bsz, seq, hdim), logits
